# baseline (speedup 1.0000x reference)
import functools

import jax
import jax.numpy as jnp
from jax import lax
from jax.experimental import pallas as pl
from jax.experimental.pallas import tpu as pltpu
from jax.experimental.pallas import tpu_sc as plsc

_TOP_K = 8
_E = 64
_H = 2048
_N = 8192
_BLK = 1024

_NC = 2
_NS = 16
_NW = _NC * _NS
_L = 16
_NCHUNK = 2
_NC2 = _N // _NCHUNK
_T = _NC2 // _NW


def _mm_body(hs_ref, w_ref, b_ref, out_ref):
    out_ref[...] = (
        lax.dot_general(w_ref[...], hs_ref[...], (((1,), (1,)), ((), ())),
                        preferred_element_type=jnp.float32)
        + b_ref[...]
    )


def _logits_t(hs, weight, bias):
    return pl.pallas_call(
        _mm_body,
        grid=(_NC2 // _BLK,),
        in_specs=[
            pl.BlockSpec((_BLK, _H), lambda i: (i, 0)),
            pl.BlockSpec((_E, _H), lambda i: (0, 0)),
            pl.BlockSpec((_E, 1), lambda i: (0, 0)),
        ],
        out_specs=pl.BlockSpec((_E, _BLK), lambda i: (0, i)),
        out_shape=jax.ShapeDtypeStruct((_E, _NC2), jnp.float32),
    )(hs, weight, bias.reshape(_E, 1))


def _merge_top16(va, ia, vb, ib):
    rv = lax.rev(vb, (0,))
    ri = lax.rev(ib, (0,))
    gt = va > rv
    hv = jnp.where(gt, va, rv)
    hi = jnp.where(gt, ia, ri)
    return plsc.sort_key_val(hv, hi, descending=True)


def _route_body(lt_hbm, probs_hbm, idx_hbm, cnt_hbm, lt_v, p_v, i_v, cnt_v):
    wid = lax.axis_index("s") * _NC + lax.axis_index("c")
    base = wid * _T
    pltpu.sync_copy(lt_hbm.at[:, pl.ds(base, _T)], lt_v)

    iota = jnp.arange(_L, dtype=jnp.int32)
    ones = jnp.ones((_L,), jnp.int32)
    neg_inf = jnp.full((_L,), -jnp.inf, jnp.float32)
    zero_i = jnp.zeros((_L,), jnp.int32)

    for c in range(_E // _L):
        cnt_v[pl.ds(c * _L, _L)] = jnp.zeros((_L,), jnp.int32)

    def group(g, carry):
        row0 = g * _L

        def insert2(e2, st):
            st = list(st)
            for u in range(2):
                e = e2 * 2 + u
                ts, ids = st[:_TOP_K], st[_TOP_K:]
                cv = lt_v[e, pl.ds(row0, _L)]
                ci = jnp.broadcast_to(e, (_L,))
                nts, nids = [], []
                for j in range(_TOP_K):
                    gt = cv > ts[j]
                    nts.append(jnp.where(gt, cv, ts[j]))
                    nids.append(jnp.where(gt, ci, ids[j]))
                    cv = jnp.where(gt, ts[j], cv)
                    ci = jnp.where(gt, ids[j], ci)
                st = nts + nids
            return tuple(st)

        st = lax.fori_loop(
            0, _E // 2, insert2,
            tuple([neg_inf] * _TOP_K) + tuple([zero_i] * _TOP_K),
        )
        ts, ids = st[:_TOP_K], st[_TOP_K:]

        nums = [jnp.exp(ts[j] - ts[0]) for j in range(_TOP_K)]
        den = nums[0]
        for j in range(1, _TOP_K):
            den = den + nums[j]
        for j in range(_TOP_K):
            p_v[pl.ds(j * _T + row0, _L)] = nums[j] / den
            i_v[pl.ds(j * _T + row0, _L)] = ids[j]
            plsc.addupdate_scatter(cnt_v, [ids[j]], ones)
        return carry

    lax.fori_loop(0, _T // _L, group, 0)

    for k in range(_TOP_K):
        pltpu.sync_copy(p_v.at[pl.ds(k * _T, _T)], probs_hbm.at[k, pl.ds(base, _T)])
        pltpu.sync_copy(i_v.at[pl.ds(k * _T, _T)], idx_hbm.at[k, pl.ds(base, _T)])
    pltpu.sync_copy(cnt_v, cnt_hbm.at[wid])


@functools.partial(
    pl.kernel,
    mesh=plsc.VectorSubcoreMesh(core_axis_name="c", subcore_axis_name="s"),
    out_type=[
        jax.ShapeDtypeStruct((_TOP_K, _NC2), jnp.float32),
        jax.ShapeDtypeStruct((_TOP_K, _NC2), jnp.int32),
        jax.ShapeDtypeStruct((_NW, _E), jnp.int32),
    ],
    scratch_types=[
        pltpu.VMEM((_E, _T), jnp.float32),
        pltpu.VMEM((_TOP_K * _T,), jnp.float32),
        pltpu.VMEM((_TOP_K * _T,), jnp.int32),
        pltpu.VMEM((_E,), jnp.int32),
    ],
    compiler_params=pltpu.CompilerParams(needs_layout_passes=False),
)
def _route(lt_hbm, probs_hbm, idx_hbm, cnt_hbm, lt_v, p_v, i_v, cnt_v):
    _route_body(lt_hbm, probs_hbm, idx_hbm, cnt_hbm, lt_v, p_v, i_v, cnt_v)


def _finish_body(p0_ref, i0_ref, p1_ref, i1_ref, pa_ref, pb_ref,
                 scores_ref, idx_ref, cnt_ref):
    i = pl.program_id(0)
    second = i >= (_NC2 // _BLK)
    p = jnp.where(second, p1_ref[...], p0_ref[...])
    ii = jnp.where(second, i1_ref[...], i0_ref[...])

    r8 = lax.broadcasted_iota(jnp.int32, (_TOP_K, _TOP_K), 0)
    c8 = lax.broadcasted_iota(jnp.int32, (_TOP_K, _TOP_K), 1)
    eye8 = (r8 == c8).astype(jnp.float32)
    dn = (((0,), (0,)), ((), ()))
    iT = lax.dot_general(ii.astype(jnp.float32), eye8, dn,
                         precision=lax.Precision.HIGHEST,
                         preferred_element_type=jnp.float32).astype(jnp.int32)
    idx_ref[...] = iT

    sub = lax.broadcasted_iota(jnp.int32, (_E, _BLK), 0)
    scT = jnp.zeros((_E, _BLK), jnp.float32)
    for k in range(_TOP_K):
        ikb = jnp.broadcast_to(ii[k:k + 1, :], (_E, _BLK))
        pkb = jnp.broadcast_to(p[k:k + 1, :], (_E, _BLK))
        scT = jnp.where(sub == ikb, pkb, scT)
    r64 = lax.broadcasted_iota(jnp.int32, (_E, _E), 0)
    c64 = lax.broadcasted_iota(jnp.int32, (_E, _E), 1)
    eye64 = (r64 == c64).astype(jnp.float32)
    scores_ref[...] = lax.dot_general(scT, eye64, dn,
                                      precision=lax.Precision.HIGHEST,
                                      preferred_element_type=jnp.float32)

    @pl.when(i == 0)
    def _():
        cnt_ref[...] = (jnp.sum(pa_ref[...], axis=0, keepdims=True)
                        + jnp.sum(pb_ref[...], axis=0, keepdims=True))


def _finish(p0, i0, p1, i1, parts0, parts1):
    half = _NC2 // _BLK
    return pl.pallas_call(
        _finish_body,
        grid=(_N // _BLK,),
        in_specs=[
            pl.BlockSpec((_TOP_K, _BLK), lambda i: (0, i % (_NC2 // _BLK))),
            pl.BlockSpec((_TOP_K, _BLK), lambda i: (0, i % (_NC2 // _BLK))),
            pl.BlockSpec((_TOP_K, _BLK), lambda i: (0, i % (_NC2 // _BLK))),
            pl.BlockSpec((_TOP_K, _BLK), lambda i: (0, i % (_NC2 // _BLK))),
            pl.BlockSpec((_NW, _E), lambda i: (0, 0)),
            pl.BlockSpec((_NW, _E), lambda i: (0, 0)),
        ],
        out_specs=[
            pl.BlockSpec((_BLK, _E), lambda i: (i, 0)),
            pl.BlockSpec((_BLK, _TOP_K), lambda i: (i, 0)),
            pl.BlockSpec((1, _E), lambda i: (0, 0)),
        ],
        out_shape=[
            jax.ShapeDtypeStruct((_N, _E), jnp.float32),
            jax.ShapeDtypeStruct((_N, _TOP_K), jnp.int32),
            jax.ShapeDtypeStruct((1, _E), jnp.int32),
        ],
        compiler_params=pltpu.CompilerParams(
            dimension_semantics=("arbitrary",),
        ),
    )(p0, i0, p1, i1, parts0, parts1)


@jax.jit
def kernel(hidden_states, weight, bias):
    hs = hidden_states.reshape(-1, _H)
    lt0 = _logits_t(hs[:_NC2], weight, bias)
    p0, i0, c0 = _route(lt0)
    lt1 = _logits_t(hs[_NC2:], weight, bias)
    p1, i1, c1 = _route(lt1)
    scores, idx, cnt = _finish(p0, i0, p1, i1, c0, c1)
    return scores, idx, cnt.reshape(_E)


# speedup vs baseline: 1.7636x; 1.7636x over previous
import functools

import jax
import jax.numpy as jnp
from jax import lax
from jax.experimental import pallas as pl
from jax.experimental.pallas import tpu as pltpu
from jax.experimental.pallas import tpu_sc as plsc

_TOP_K = 8
_E = 64
_H = 2048
_N = 8192
_BLK = 1024

_NC = 2
_NS = 16
_NW = _NC * _NS
_L = 16
_T = _N // _NW
_G = _T // _L


def _mm_body(hs_ref, w_ref, b_ref, out_ref):
    out_ref[...] = (
        lax.dot_general(w_ref[...], hs_ref[...], (((1,), (1,)), ((), ())),
                        preferred_element_type=jnp.float32)
        + b_ref[...]
    )


def _logits_t(hs, weight, bias):
    return pl.pallas_call(
        _mm_body,
        grid=(_N // _BLK,),
        in_specs=[
            pl.BlockSpec((_BLK, _H), lambda i: (i, 0)),
            pl.BlockSpec((_E, _H), lambda i: (0, 0)),
            pl.BlockSpec((_E, 1), lambda i: (0, 0)),
        ],
        out_specs=pl.BlockSpec((_E, _BLK), lambda i: (0, i)),
        out_shape=jax.ShapeDtypeStruct((_E, _N), jnp.float32),
    )(hs, weight, bias.reshape(_E, 1))


def _route_body(lt_hbm, scores_hbm, idx_hbm, cnt_hbm, lt_v, sc_v, idx_v, cnt_v):
    wid = lax.axis_index("s") * _NC + lax.axis_index("c")
    base = wid * _T
    pltpu.sync_copy(lt_hbm.at[:, pl.ds(base, _T)], lt_v)

    iota = jnp.arange(_L, dtype=jnp.int32)
    zeros = jnp.zeros((_L,), jnp.float32)
    ones = jnp.ones((_L,), jnp.int32)
    neg_inf = jnp.full((_L,), -jnp.inf, jnp.float32)

    for c in range(_E // _L):
        cnt_v[pl.ds(c * _L, _L)] = jnp.zeros((_L,), jnp.int32)

    def group(g, carry):
        row0 = g * _L

        def insert(e, st):
            ts, ids = st[:_TOP_K], st[_TOP_K:]
            cv = lt_v[e, pl.ds(row0, _L)]
            ci = jnp.broadcast_to(e, (_L,))
            nts, nids = [], []
            for j in range(_TOP_K):
                gt = cv > ts[j]
                nts.append(jnp.where(gt, cv, ts[j]))
                nids.append(jnp.where(gt, ci, ids[j]))
                cv = jnp.where(gt, ts[j], cv)
                ci = jnp.where(gt, ids[j], ci)
            return tuple(nts) + tuple(nids)

        st = lax.fori_loop(
            0, _E, insert,
            tuple([neg_inf] * _TOP_K) + tuple([jnp.zeros((_L,), jnp.int32)] * _TOP_K),
        )
        ts, ids = st[:_TOP_K], st[_TOP_K:]

        nums = [jnp.exp(ts[j] - ts[0]) for j in range(_TOP_K)]
        den = nums[0]
        for j in range(1, _TOP_K):
            den = den + nums[j]
        rden = jnp.float32(1.0) / den

        for rc in range(_L * _E // _L):
            sc_v[pl.ds(row0 * _E + rc * _L, _L)] = zeros

        rows = row0 + iota
        for j in range(_TOP_K):
            plsc.store_scatter(sc_v, [rows * _E + ids[j]], nums[j] * rden)
            plsc.store_scatter(idx_v, [rows * _TOP_K + j], ids[j])
            plsc.addupdate_scatter(cnt_v, [ids[j]], ones)
        return carry

    lax.fori_loop(0, _G, group, 0)

    pltpu.sync_copy(sc_v, scores_hbm.at[pl.ds(base * _E, _T * _E)])
    pltpu.sync_copy(idx_v, idx_hbm.at[pl.ds(base * _TOP_K, _T * _TOP_K)])
    pltpu.sync_copy(cnt_v, cnt_hbm.at[wid])


@functools.partial(
    pl.kernel,
    mesh=plsc.VectorSubcoreMesh(core_axis_name="c", subcore_axis_name="s"),
    out_type=[
        jax.ShapeDtypeStruct((_N * _E,), jnp.float32),
        jax.ShapeDtypeStruct((_N * _TOP_K,), jnp.int32),
        jax.ShapeDtypeStruct((_NW, _E), jnp.int32),
    ],
    scratch_types=[
        pltpu.VMEM((_E, _T), jnp.float32),
        pltpu.VMEM((_T * _E,), jnp.float32),
        pltpu.VMEM((_T * _TOP_K,), jnp.int32),
        pltpu.VMEM((_E,), jnp.int32),
    ],
    compiler_params=pltpu.CompilerParams(needs_layout_passes=False),
)
def _route(lt_hbm, scores_hbm, idx_hbm, cnt_hbm, lt_v, sc_v, idx_v, cnt_v):
    _route_body(lt_hbm, scores_hbm, idx_hbm, cnt_hbm, lt_v, sc_v, idx_v, cnt_v)


def _cnt_body(parts_ref, out_ref):
    out_ref[...] = jnp.sum(parts_ref[...], axis=0, keepdims=True)


def _cnt_reduce(parts):
    return pl.pallas_call(
        _cnt_body,
        out_shape=jax.ShapeDtypeStruct((1, _E), jnp.int32),
    )(parts)


@jax.jit
def kernel(hidden_states, weight, bias):
    hs = hidden_states.reshape(-1, _H)
    lt = _logits_t(hs, weight, bias)
    scores, idx, cnt_parts = _route(lt)
    cnt = _cnt_reduce(cnt_parts)
    return scores.reshape(_N, _E), idx.reshape(_N, _TOP_K), cnt.reshape(_E)


# speedup vs baseline: 1.7738x; 1.0058x over previous
import functools

import jax
import jax.numpy as jnp
from jax import lax
from jax.experimental import pallas as pl
from jax.experimental.pallas import tpu as pltpu
from jax.experimental.pallas import tpu_sc as plsc

_TOP_K = 8
_E = 64
_H = 2048
_N = 8192
_BLK = 1024

_NC = 2
_NS = 16
_NW = _NC * _NS
_L = 16
_T = _N // _NW
_G = _T // _L


def _mm_body(hs_ref, w_ref, b_ref, out_ref):
    out_ref[...] = (
        lax.dot_general(w_ref[...], hs_ref[...], (((1,), (1,)), ((), ())),
                        preferred_element_type=jnp.float32)
        + b_ref[...]
    )


def _logits_t(hs, weight, bias):
    return pl.pallas_call(
        _mm_body,
        grid=(_N // _BLK,),
        in_specs=[
            pl.BlockSpec((_BLK, _H), lambda i: (i, 0)),
            pl.BlockSpec((_E, _H), lambda i: (0, 0)),
            pl.BlockSpec((_E, 1), lambda i: (0, 0)),
        ],
        out_specs=pl.BlockSpec((_E, _BLK), lambda i: (0, i)),
        out_shape=jax.ShapeDtypeStruct((_E, _N), jnp.float32),
    )(hs, weight, bias.reshape(_E, 1))


def _route_body(lt_hbm, scores_hbm, idx_hbm, cnt_hbm, lt_v, sc_v, idx_v, cnt_v):
    wid = lax.axis_index("s") * _NC + lax.axis_index("c")
    base = wid * _T
    pltpu.sync_copy(lt_hbm.at[:, pl.ds(base, _T)], lt_v)

    iota = jnp.arange(_L, dtype=jnp.int32)
    zeros = jnp.zeros((_L,), jnp.float32)
    ones = jnp.ones((_L,), jnp.int32)
    neg_inf = jnp.full((_L,), -jnp.inf, jnp.float32)

    for c in range(_E // _L):
        cnt_v[pl.ds(c * _L, _L)] = jnp.zeros((_L,), jnp.int32)

    def group(g, carry):
        row0 = g * _L

        def insert2(e2, st):
            st = list(st)
            for u in range(2):
                e = e2 * 2 + u
                ts, ids = st[:_TOP_K], st[_TOP_K:]
                cv = lt_v[e, pl.ds(row0, _L)]
                ci = jnp.broadcast_to(e, (_L,))
                nts, nids = [], []
                for j in range(_TOP_K):
                    gt = cv > ts[j]
                    nts.append(jnp.where(gt, cv, ts[j]))
                    nids.append(jnp.where(gt, ci, ids[j]))
                    cv = jnp.where(gt, ts[j], cv)
                    ci = jnp.where(gt, ids[j], ci)
                st = nts + nids
            return tuple(st)

        st = lax.fori_loop(
            0, _E // 2, insert2,
            tuple([neg_inf] * _TOP_K) + tuple([jnp.zeros((_L,), jnp.int32)] * _TOP_K),
        )
        ts, ids = st[:_TOP_K], st[_TOP_K:]

        nums = [jnp.exp(ts[j] - ts[0]) for j in range(_TOP_K)]
        den = nums[0]
        for j in range(1, _TOP_K):
            den = den + nums[j]
        rden = jnp.float32(1.0) / den

        for rc in range(_L * _E // _L):
            sc_v[pl.ds(row0 * _E + rc * _L, _L)] = zeros

        rows = row0 + iota
        for j in range(_TOP_K):
            plsc.store_scatter(sc_v, [rows * _E + ids[j]], nums[j] * rden)
            plsc.store_scatter(idx_v, [rows * _TOP_K + j], ids[j])
            plsc.addupdate_scatter(cnt_v, [ids[j]], ones)
        return carry

    lax.fori_loop(0, _G, group, 0)

    pltpu.sync_copy(sc_v, scores_hbm.at[pl.ds(base * _E, _T * _E)])
    pltpu.sync_copy(idx_v, idx_hbm.at[pl.ds(base * _TOP_K, _T * _TOP_K)])
    pltpu.sync_copy(cnt_v, cnt_hbm.at[wid])


@functools.partial(
    pl.kernel,
    mesh=plsc.VectorSubcoreMesh(core_axis_name="c", subcore_axis_name="s"),
    out_type=[
        jax.ShapeDtypeStruct((_N * _E,), jnp.float32),
        jax.ShapeDtypeStruct((_N * _TOP_K,), jnp.int32),
        jax.ShapeDtypeStruct((_NW, _E), jnp.int32),
    ],
    scratch_types=[
        pltpu.VMEM((_E, _T), jnp.float32),
        pltpu.VMEM((_T * _E,), jnp.float32),
        pltpu.VMEM((_T * _TOP_K,), jnp.int32),
        pltpu.VMEM((_E,), jnp.int32),
    ],
    compiler_params=pltpu.CompilerParams(needs_layout_passes=False),
)
def _route(lt_hbm, scores_hbm, idx_hbm, cnt_hbm, lt_v, sc_v, idx_v, cnt_v):
    _route_body(lt_hbm, scores_hbm, idx_hbm, cnt_hbm, lt_v, sc_v, idx_v, cnt_v)


def _cnt_body(parts_ref, out_ref):
    out_ref[...] = jnp.sum(parts_ref[...], axis=0, keepdims=True)


def _cnt_reduce(parts):
    return pl.pallas_call(
        _cnt_body,
        out_shape=jax.ShapeDtypeStruct((1, _E), jnp.int32),
    )(parts)


@jax.jit
def kernel(hidden_states, weight, bias):
    hs = hidden_states.reshape(-1, _H)
    lt = _logits_t(hs, weight, bias)
    scores, idx, cnt_parts = _route(lt)
    cnt = _cnt_reduce(cnt_parts)
    return scores.reshape(_N, _E), idx.reshape(_N, _TOP_K), cnt.reshape(_E)
